# baseline (speedup 1.0000x reference)
import functools

import jax
import jax.numpy as jnp
from jax import lax
from jax.experimental import pallas as pl
from jax.experimental.pallas import tpu as pltpu
from jax.experimental.pallas import tpu_sc as plsc

N = 10000
E = 160000
H = 256
R = 8
NH = 8
HD = H // NH


_RB = 1024


def _proj_body(x_ref, wq_ref, wk_ref, wv_ref, bq_ref, bk_ref, bv_ref,
               q_ref, k_ref, v_ref):
    x = x_ref[...]
    q_ref[...] = jnp.dot(x, wq_ref[...], preferred_element_type=jnp.float32) + bq_ref[...]
    k_ref[...] = jnp.dot(x, wk_ref[...], preferred_element_type=jnp.float32) + bk_ref[...]
    v_ref[...] = jnp.dot(x, wv_ref[...], preferred_element_type=jnp.float32) + bv_ref[...]


def _projections(x, wq, wk, wv, bq, bk, bv):
    grid = (pl.cdiv(N, _RB),)
    return pl.pallas_call(
        _proj_body,
        grid=grid,
        in_specs=[
            pl.BlockSpec((_RB, H), lambda i: (i, 0)),
            pl.BlockSpec((H, H), lambda i: (0, 0)),
            pl.BlockSpec((H, R * H), lambda i: (0, 0)),
            pl.BlockSpec((H, R * H), lambda i: (0, 0)),
            pl.BlockSpec((1, H), lambda i: (0, 0)),
            pl.BlockSpec((1, R * H), lambda i: (0, 0)),
            pl.BlockSpec((1, R * H), lambda i: (0, 0)),
        ],
        out_specs=[
            pl.BlockSpec((_RB, H), lambda i: (i, 0)),
            pl.BlockSpec((_RB, R * H), lambda i: (i, 0)),
            pl.BlockSpec((_RB, R * H), lambda i: (i, 0)),
        ],
        out_shape=[
            jax.ShapeDtypeStruct((N, H), jnp.float32),
            jax.ShapeDtypeStruct((N, R * H), jnp.float32),
            jax.ShapeDtypeStruct((N, R * H), jnp.float32),
        ],
    )(x, wq, wk, wv, bq, bk, bv)


_GB = 128
_NW = 32
_EP = 163840
_NGB = _EP // _GB
_GIT = _NGB // _NW


def _gather_sc(qtab, ktab, vtab, src, dst, et):
    mesh = plsc.VectorSubcoreMesh(core_axis_name="c", subcore_axis_name="s")

    @functools.partial(
        pl.kernel, mesh=mesh,
        out_type=[
            jax.ShapeDtypeStruct((_EP, H), jnp.float32),
            jax.ShapeDtypeStruct((_EP, H), jnp.float32),
            jax.ShapeDtypeStruct((_EP, H), jnp.float32),
        ],
        scratch_types=[
            pltpu.VMEM((_GB,), jnp.int32),
            pltpu.VMEM((_GB,), jnp.int32),
            pltpu.VMEM((_GB,), jnp.int32),
            pltpu.VMEM((_GB,), jnp.int32),
            pltpu.VMEM((_GB, H), jnp.float32),
            pltpu.VMEM((_GB, H), jnp.float32),
            pltpu.VMEM((_GB, H), jnp.float32),
            pltpu.SemaphoreType.DMA,
            pltpu.SemaphoreType.DMA,
            pltpu.SemaphoreType.DMA,
        ],
        compiler_params=pltpu.CompilerParams(needs_layout_passes=False),
    )
    def k(qtab_h, ktab_h, vtab_h, src_h, dst_h, et_h, qg_h, kg_h, vg_h,
          sbuf, dbuf, ebuf, kibuf, qstage, kstage, vstage, sem0, sem1, sem2):
        wid = lax.axis_index("s") * 2 + lax.axis_index("c")

        def body(i, _):
            b = wid + i * _NW
            if True:
                e0 = b * _GB
                pltpu.sync_copy(src_h.at[pl.ds(e0, _GB)], sbuf)
                pltpu.sync_copy(dst_h.at[pl.ds(e0, _GB)], dbuf)
                pltpu.sync_copy(et_h.at[pl.ds(e0, _GB)], ebuf)
                for g in range(_GB // 16):
                    sl = pl.ds(g * 16, 16)
                    kibuf[sl] = sbuf[sl] * R + ebuf[sl]
                cq = pltpu.async_copy(qtab_h.at[dbuf], qstage, sem0)
                ck = pltpu.async_copy(ktab_h.at[kibuf], kstage, sem1)
                cv = pltpu.async_copy(vtab_h.at[kibuf], vstage, sem2)
                cq.wait()
                ck.wait()
                cv.wait()
                pltpu.sync_copy(qstage, qg_h.at[pl.ds(e0, _GB)])
                pltpu.sync_copy(kstage, kg_h.at[pl.ds(e0, _GB)])
                pltpu.sync_copy(vstage, vg_h.at[pl.ds(e0, _GB)])

            return _

        lax.fori_loop(0, _GIT, body, None)

    return k(qtab, ktab, vtab, src, dst, et)


_EB = 5120
_NEB = _EP // _EB
_NEG = -1e30


def _s1_body(qg_ref, kg_ref, et_ref, b8_ref, st_ref, pmax_ref):
    qk = qg_ref[...] * kg_ref[...]
    st = lax.dot_general(b8_ref[...], qk, (((1,), (1,)), ((), ())),
                         preferred_element_type=jnp.float32)
    st_ref[...] = st
    et = et_ref[...]
    for r in range(R):
        m = jnp.max(jnp.where(et == r, st, _NEG), axis=1)
        pmax_ref[0, r, :] = m


def _s2_body(st_ref, et_ref, pmax_ref, pt_ref, psum_ref):
    mx = jnp.max(pmax_ref[...], axis=0)
    et = et_ref[...]
    st = st_ref[...]
    msel = jnp.zeros_like(st)
    for r in range(R):
        msel = jnp.where(et == r, mx[r].reshape(NH, 1), msel)
    p = jnp.exp(st - msel)
    pt_ref[...] = p
    for r in range(R):
        s = jnp.sum(jnp.where(et == r, p, 0.0), axis=1)
        psum_ref[0, r, :] = s


def _s3_body(pt_ref, et_ref, psum_ref, vg_ref, b8_ref, msg_ref):
    denom = jnp.sum(psum_ref[...], axis=0)
    inv = 1.0 / denom
    et = et_ref[...]
    p = pt_ref[...]
    isel = jnp.zeros_like(p)
    for r in range(R):
        isel = jnp.where(et == r, inv[r].reshape(NH, 1), isel)
    wt = p * isel
    wide = lax.dot_general(wt, b8_ref[...], (((0,), (0,)), ((), ())),
                           preferred_element_type=jnp.float32)
    msg_ref[...] = jnp.transpose(wide * vg_ref[...])


def _scores_and_messages(qg, kg, vg, et2d, b8):
    st, pmax = pl.pallas_call(
        _s1_body,
        grid=(_NEB,),
        in_specs=[
            pl.BlockSpec((_EB, H), lambda i: (i, 0)),
            pl.BlockSpec((_EB, H), lambda i: (i, 0)),
            pl.BlockSpec((1, _EB), lambda i: (0, i)),
            pl.BlockSpec((NH, H), lambda i: (0, 0)),
        ],
        out_specs=[
            pl.BlockSpec((NH, _EB), lambda i: (0, i)),
            pl.BlockSpec((1, R, NH), lambda i: (i, 0, 0)),
        ],
        out_shape=[
            jax.ShapeDtypeStruct((NH, _EP), jnp.float32),
            jax.ShapeDtypeStruct((_NEB, R, NH), jnp.float32),
        ],
    )(qg, kg, et2d, b8)

    pt, psum = pl.pallas_call(
        _s2_body,
        grid=(_NEB,),
        in_specs=[
            pl.BlockSpec((NH, _EB), lambda i: (0, i)),
            pl.BlockSpec((1, _EB), lambda i: (0, i)),
            pl.BlockSpec((_NEB, R, NH), lambda i: (0, 0, 0)),
        ],
        out_specs=[
            pl.BlockSpec((NH, _EB), lambda i: (0, i)),
            pl.BlockSpec((1, R, NH), lambda i: (i, 0, 0)),
        ],
        out_shape=[
            jax.ShapeDtypeStruct((NH, _EP), jnp.float32),
            jax.ShapeDtypeStruct((_NEB, R, NH), jnp.float32),
        ],
    )(st, et2d, pmax)

    msg = pl.pallas_call(
        _s3_body,
        grid=(_NEB,),
        in_specs=[
            pl.BlockSpec((NH, _EB), lambda i: (0, i)),
            pl.BlockSpec((1, _EB), lambda i: (0, i)),
            pl.BlockSpec((_NEB, R, NH), lambda i: (0, 0, 0)),
            pl.BlockSpec((_EB, H), lambda i: (i, 0)),
            pl.BlockSpec((NH, H), lambda i: (0, 0)),
        ],
        out_specs=pl.BlockSpec((H, _EB), lambda i: (0, i)),
        out_shape=jax.ShapeDtypeStruct((H, _EP), jnp.float32),
    )(pt, et2d, psum, vg, b8)
    return msg


_SB = 1280
_NSB = _EP // _SB
_CPT = H // _NW


def _scatter_sc(msg, dst, z8, z1):
    mesh = plsc.VectorSubcoreMesh(core_axis_name="c", subcore_axis_name="s")

    @functools.partial(
        pl.kernel, mesh=mesh,
        out_type=jax.ShapeDtypeStruct((_NW * N * _CPT,), jnp.float32),
        scratch_types=[
            pltpu.VMEM((N * _CPT,), jnp.float32),
            pltpu.VMEM((N,), jnp.float32),
            pltpu.VMEM((_SB,), jnp.int32),
            pltpu.VMEM((_CPT, _SB), jnp.float32),
            pltpu.VMEM((_SB,), jnp.int32),
            pltpu.VMEM((_CPT, _SB), jnp.float32),
            pltpu.SemaphoreType.DMA,
            pltpu.SemaphoreType.DMA,
            pltpu.SemaphoreType.DMA,
            pltpu.SemaphoreType.DMA,
        ],
        compiler_params=pltpu.CompilerParams(needs_layout_passes=False),
    )
    def k(msg_h, dst_h, zA_h, z1_h, agg_h, aggl, cntl,
          dbuf0, mbuf0, dbuf1, mbuf1, sd0, sm0, sd1, sm1):
        wid = lax.axis_index("s") * 2 + lax.axis_index("c")
        ones16 = jnp.full((16,), 1.0, jnp.float32)
        pltpu.sync_copy(zA_h, aggl)
        pltpu.sync_copy(z1_h, cntl)

        def refs(b, dbuf, mbuf):
            e0 = b * _SB
            return (dst_h.at[pl.ds(e0, _SB)], dbuf,
                    msg_h.at[pl.ds(wid * _CPT, _CPT), pl.ds(e0, _SB)], mbuf)

        def issue(b, dbuf, mbuf, sd, sm):
            dsrc, ddst, msrc, mdst = refs(b, dbuf, mbuf)
            pltpu.async_copy(dsrc, ddst, sd)
            pltpu.async_copy(msrc, mdst, sm)

        def wait(b, dbuf, mbuf, sd, sm):
            dsrc, ddst, msrc, mdst = refs(b, dbuf, mbuf)
            pltpu.make_async_copy(dsrc, ddst, sd).wait()
            pltpu.make_async_copy(msrc, mdst, sm).wait()

        def process(dbuf, mbuf):
            for g in range(_SB // 16):
                sl = pl.ds(g * 16, 16)
                d16 = dbuf[sl]
                plsc.addupdate_scatter(cntl, [d16], ones16)
                fbase = d16 * _CPT
                for c in range(_CPT):
                    plsc.addupdate_scatter(aggl, [fbase + c], mbuf[c, sl])

        issue(0, dbuf0, mbuf0, sd0, sm0)

        def body(i, _):
            b0 = 2 * i
            issue(b0 + 1, dbuf1, mbuf1, sd1, sm1)
            wait(b0, dbuf0, mbuf0, sd0, sm0)
            process(dbuf0, mbuf0)

            @pl.when(b0 + 2 < _NSB)
            def _issue_next():
                issue(b0 + 2, dbuf0, mbuf0, sd0, sm0)

            wait(b0 + 1, dbuf1, mbuf1, sd1, sm1)
            process(dbuf1, mbuf1)
            return _

        lax.fori_loop(0, _NSB // 2, body, None)

        pad_fix = jnp.where(lax.iota(jnp.int32, 16) == 0,
                            jnp.float32(_EP - E), jnp.float32(0.0))
        cntl[pl.ds(0, 16)] = cntl[pl.ds(0, 16)] - pad_fix

        def fin(j, _):
            sl = pl.ds(j * 16 * _CPT, 16)
            inv = 1.0 / jnp.maximum(cntl[pl.ds(j * 16, 16)], 1.0)
            fbase = (lax.iota(jnp.int32, 16) + j * 16) * _CPT
            for c in range(_CPT):
                v = plsc.load_gather(aggl, [fbase + c])
                plsc.store_scatter(aggl, [fbase + c], v * inv)
            return _

        lax.fori_loop(0, N // 16, fin, None)
        pltpu.sync_copy(aggl, agg_h.at[pl.ds(wid * (N * _CPT), N * _CPT)])

    return k(msg, dst, z8, z1)


_FB = 512


def _final_body(agg_ref, nf_ref, owt_ref, ob_ref, w1t_ref, b1_ref,
                w2t_ref, b2_ref, g1_ref, be1_ref, g2_ref, be2_ref, out_ref):
    acc = jnp.zeros((_FB, H), jnp.float32)
    for t in range(_NW):
        acc = acc + jnp.dot(agg_ref[t], owt_ref[t],
                            preferred_element_type=jnp.float32)
    x1 = nf_ref[...] + acc + ob_ref[...]
    mu = jnp.mean(x1, axis=1, keepdims=True)
    var = jnp.mean((x1 - mu) ** 2, axis=1, keepdims=True)
    x = (x1 - mu) * lax.rsqrt(var + 1e-5) * g1_ref[...] + be1_ref[...]
    h = jnp.dot(x, w1t_ref[...], preferred_element_type=jnp.float32) + b1_ref[...]
    h = 0.5 * h * (1.0 + lax.erf(h * (2.0 ** -0.5)))
    y = jnp.dot(h, w2t_ref[...], preferred_element_type=jnp.float32) + b2_ref[...]
    x2 = x + y
    mu2 = jnp.mean(x2, axis=1, keepdims=True)
    var2 = jnp.mean((x2 - mu2) ** 2, axis=1, keepdims=True)
    out_ref[...] = (x2 - mu2) * lax.rsqrt(var2 + 1e-5) * g2_ref[...] + be2_ref[...]


def _final(agg, nf, owt, ob, w1t, b1, w2t, b2, g1, be1, g2, be2):
    grid = (pl.cdiv(N, _FB),)
    return pl.pallas_call(
        _final_body,
        grid=grid,
        in_specs=[
            pl.BlockSpec((_NW, _FB, _CPT), lambda i: (0, i, 0)),
            pl.BlockSpec((_FB, H), lambda i: (i, 0)),
            pl.BlockSpec((_NW, _CPT, H), lambda i: (0, 0, 0)),
            pl.BlockSpec((1, H), lambda i: (0, 0)),
            pl.BlockSpec((H, 2 * H), lambda i: (0, 0)),
            pl.BlockSpec((1, 2 * H), lambda i: (0, 0)),
            pl.BlockSpec((2 * H, H), lambda i: (0, 0)),
            pl.BlockSpec((1, H), lambda i: (0, 0)),
            pl.BlockSpec((1, H), lambda i: (0, 0)),
            pl.BlockSpec((1, H), lambda i: (0, 0)),
            pl.BlockSpec((1, H), lambda i: (0, 0)),
            pl.BlockSpec((1, H), lambda i: (0, 0)),
        ],
        out_specs=pl.BlockSpec((_FB, H), lambda i: (i, 0)),
        out_shape=jax.ShapeDtypeStruct((N, H), jnp.float32),
    )(agg, nf, owt, ob, w1t, b1, w2t, b2, g1, be1, g2, be2)


def kernel(node_feats, edge_index, edge_type, qW, qb, kW, kb, vW, vb,
           oW, ob, w1, b1, w2, b2, g1, be1, g2, be2):
    scale = HD ** (-0.5)
    src = edge_index[0]
    dst = edge_index[1]
    et = edge_type
    zpad = jnp.zeros((_EP - E,), jnp.int32)
    srcp = jnp.concatenate([src, zpad])
    dstp = jnp.concatenate([dst, zpad])
    etp0 = jnp.concatenate([et, zpad])
    et2d = jnp.concatenate([et, zpad + R]).reshape(1, _EP)

    wq = qW.T * scale
    bq = (qb * scale).reshape(1, H)
    wk = jnp.transpose(kW, (2, 0, 1)).reshape(H, R * H)
    wv = jnp.transpose(vW, (2, 0, 1)).reshape(H, R * H)
    bk = kb.reshape(1, R * H)
    bv = vb.reshape(1, R * H)
    b8 = (jnp.arange(H, dtype=jnp.int32)[None, :] // HD ==
          jnp.arange(NH, dtype=jnp.int32)[:, None]).astype(jnp.float32)
    owt = oW.T.reshape(_NW, _CPT, H)
    w1t = w1.T
    w2t = w2.T
    z8 = jnp.zeros((N * _CPT,), jnp.float32)
    z1 = jnp.zeros((N,), jnp.float32)

    qtab, ktab2, vtab2 = _projections(node_feats, wq, wk, wv, bq, bk, bv)
    ktab = ktab2.reshape(N * R, H)
    vtab = vtab2.reshape(N * R, H)

    qg, kg, vg = _gather_sc(qtab, ktab, vtab, srcp, dstp, etp0)
    msg = _scores_and_messages(qg, kg, vg, et2d, b8)
    agg = _scatter_sc(msg, dstp, z8, z1).reshape(_NW, N, _CPT)

    return _final(agg, node_feats, owt, ob.reshape(1, H), w1t,
                  b1.reshape(1, 2 * H), w2t, b2.reshape(1, H),
                  g1.reshape(1, H), be1.reshape(1, H),
                  g2.reshape(1, H), be2.reshape(1, H))


# speedup vs baseline: 1.1529x; 1.1529x over previous
import functools

import jax
import jax.numpy as jnp
from jax import lax
from jax.experimental import pallas as pl
from jax.experimental.pallas import tpu as pltpu
from jax.experimental.pallas import tpu_sc as plsc

N = 10000
E = 160000
H = 256
R = 8
NH = 8
HD = H // NH


_RB = 1024


def _proj_body(x_ref, wq_ref, wk_ref, wv_ref, bq_ref, bk_ref, bv_ref,
               q_ref, k_ref, v_ref):
    x = x_ref[...]
    q_ref[...] = jnp.dot(x, wq_ref[...], preferred_element_type=jnp.float32) + bq_ref[...]
    k_ref[...] = jnp.dot(x, wk_ref[...], preferred_element_type=jnp.float32) + bk_ref[...]
    v_ref[...] = jnp.dot(x, wv_ref[...], preferred_element_type=jnp.float32) + bv_ref[...]


def _projections(x, wq, wk, wv, bq, bk, bv):
    grid = (pl.cdiv(N, _RB),)
    return pl.pallas_call(
        _proj_body,
        grid=grid,
        in_specs=[
            pl.BlockSpec((_RB, H), lambda i: (i, 0)),
            pl.BlockSpec((H, H), lambda i: (0, 0)),
            pl.BlockSpec((H, R * H), lambda i: (0, 0)),
            pl.BlockSpec((H, R * H), lambda i: (0, 0)),
            pl.BlockSpec((1, H), lambda i: (0, 0)),
            pl.BlockSpec((1, R * H), lambda i: (0, 0)),
            pl.BlockSpec((1, R * H), lambda i: (0, 0)),
        ],
        out_specs=[
            pl.BlockSpec((_RB, H), lambda i: (i, 0)),
            pl.BlockSpec((_RB, R * H), lambda i: (i, 0)),
            pl.BlockSpec((_RB, R * H), lambda i: (i, 0)),
        ],
        out_shape=[
            jax.ShapeDtypeStruct((N, H), jnp.float32),
            jax.ShapeDtypeStruct((N, R * H), jnp.float32),
            jax.ShapeDtypeStruct((N, R * H), jnp.float32),
        ],
    )(x, wq, wk, wv, bq, bk, bv)


_GB = 128
_NW = 32
_EP = 163840
_NGB = _EP // _GB
_GIT = _NGB // _NW


def _gather_sc(qtab, ktab, vtab, src, dst, et):
    mesh = plsc.VectorSubcoreMesh(core_axis_name="c", subcore_axis_name="s")

    @functools.partial(
        pl.kernel, mesh=mesh,
        out_type=[
            jax.ShapeDtypeStruct((_EP, H), jnp.float32),
            jax.ShapeDtypeStruct((_EP, H), jnp.float32),
            jax.ShapeDtypeStruct((_EP, H), jnp.float32),
        ],
        scratch_types=[
            pltpu.VMEM((_GB,), jnp.int32),
            pltpu.VMEM((_GB,), jnp.int32),
            pltpu.VMEM((_GB,), jnp.int32),
            pltpu.VMEM((_GB,), jnp.int32),
            pltpu.VMEM((_GB, H), jnp.float32),
            pltpu.VMEM((_GB, H), jnp.float32),
            pltpu.VMEM((_GB, H), jnp.float32),
            pltpu.SemaphoreType.DMA,
            pltpu.SemaphoreType.DMA,
            pltpu.SemaphoreType.DMA,
        ],
        compiler_params=pltpu.CompilerParams(needs_layout_passes=False),
    )
    def k(qtab_h, ktab_h, vtab_h, src_h, dst_h, et_h, qg_h, kg_h, vg_h,
          sbuf, dbuf, ebuf, kibuf, qstage, kstage, vstage, sem0, sem1, sem2):
        wid = lax.axis_index("s") * 2 + lax.axis_index("c")

        def body(i, _):
            b = wid + i * _NW
            if True:
                e0 = b * _GB
                pltpu.sync_copy(src_h.at[pl.ds(e0, _GB)], sbuf)
                pltpu.sync_copy(dst_h.at[pl.ds(e0, _GB)], dbuf)
                pltpu.sync_copy(et_h.at[pl.ds(e0, _GB)], ebuf)
                for g in range(_GB // 16):
                    sl = pl.ds(g * 16, 16)
                    kibuf[sl] = sbuf[sl] * R + ebuf[sl]
                cq = pltpu.async_copy(qtab_h.at[dbuf], qstage, sem0)
                ck = pltpu.async_copy(ktab_h.at[kibuf], kstage, sem1)
                cv = pltpu.async_copy(vtab_h.at[kibuf], vstage, sem2)
                cq.wait()
                ck.wait()
                cv.wait()
                pltpu.sync_copy(qstage, qg_h.at[pl.ds(e0, _GB)])
                pltpu.sync_copy(kstage, kg_h.at[pl.ds(e0, _GB)])
                pltpu.sync_copy(vstage, vg_h.at[pl.ds(e0, _GB)])

            return _

        lax.fori_loop(0, _GIT, body, None)

    return k(qtab, ktab, vtab, src, dst, et)


_EB = 5120
_NEB = _EP // _EB
_NEG = -1e30


def _s1_body(qg_ref, kg_ref, et_ref, b8_ref, st_ref, pmax_ref):
    qk = qg_ref[...] * kg_ref[...]
    st = lax.dot_general(b8_ref[...], qk, (((1,), (1,)), ((), ())),
                         preferred_element_type=jnp.float32)
    st_ref[...] = st
    et = et_ref[...]
    for r in range(R):
        m = jnp.max(jnp.where(et == r, st, _NEG), axis=1)
        pmax_ref[0, r, :] = m


def _s2_body(st_ref, et_ref, pmax_ref, pt_ref, psum_ref):
    mx = jnp.max(pmax_ref[...], axis=0)
    et = et_ref[...]
    st = st_ref[...]
    msel = jnp.zeros_like(st)
    for r in range(R):
        msel = jnp.where(et == r, mx[r].reshape(NH, 1), msel)
    p = jnp.exp(st - msel)
    pt_ref[...] = p
    for r in range(R):
        s = jnp.sum(jnp.where(et == r, p, 0.0), axis=1)
        psum_ref[0, r, :] = s


def _s3_body(pt_ref, et_ref, psum_ref, vg_ref, b8_ref, msg_ref):
    denom = jnp.sum(psum_ref[...], axis=0)
    inv = 1.0 / denom
    et = et_ref[...]
    p = pt_ref[...]
    isel = jnp.zeros_like(p)
    for r in range(R):
        isel = jnp.where(et == r, inv[r].reshape(NH, 1), isel)
    wt = p * isel
    wide = lax.dot_general(wt, b8_ref[...], (((0,), (0,)), ((), ())),
                           preferred_element_type=jnp.float32)
    msg_ref[...] = jnp.transpose(wide * vg_ref[...])


def _scores_and_messages(qg, kg, vg, et2d, b8):
    st, pmax = pl.pallas_call(
        _s1_body,
        grid=(_NEB,),
        in_specs=[
            pl.BlockSpec((_EB, H), lambda i: (i, 0)),
            pl.BlockSpec((_EB, H), lambda i: (i, 0)),
            pl.BlockSpec((1, _EB), lambda i: (0, i)),
            pl.BlockSpec((NH, H), lambda i: (0, 0)),
        ],
        out_specs=[
            pl.BlockSpec((NH, _EB), lambda i: (0, i)),
            pl.BlockSpec((1, R, NH), lambda i: (i, 0, 0)),
        ],
        out_shape=[
            jax.ShapeDtypeStruct((NH, _EP), jnp.float32),
            jax.ShapeDtypeStruct((_NEB, R, NH), jnp.float32),
        ],
    )(qg, kg, et2d, b8)

    pt, psum = pl.pallas_call(
        _s2_body,
        grid=(_NEB,),
        in_specs=[
            pl.BlockSpec((NH, _EB), lambda i: (0, i)),
            pl.BlockSpec((1, _EB), lambda i: (0, i)),
            pl.BlockSpec((_NEB, R, NH), lambda i: (0, 0, 0)),
        ],
        out_specs=[
            pl.BlockSpec((NH, _EB), lambda i: (0, i)),
            pl.BlockSpec((1, R, NH), lambda i: (i, 0, 0)),
        ],
        out_shape=[
            jax.ShapeDtypeStruct((NH, _EP), jnp.float32),
            jax.ShapeDtypeStruct((_NEB, R, NH), jnp.float32),
        ],
    )(st, et2d, pmax)

    msg = pl.pallas_call(
        _s3_body,
        grid=(_NEB,),
        in_specs=[
            pl.BlockSpec((NH, _EB), lambda i: (0, i)),
            pl.BlockSpec((1, _EB), lambda i: (0, i)),
            pl.BlockSpec((_NEB, R, NH), lambda i: (0, 0, 0)),
            pl.BlockSpec((_EB, H), lambda i: (i, 0)),
            pl.BlockSpec((NH, H), lambda i: (0, 0)),
        ],
        out_specs=pl.BlockSpec((H, _EB), lambda i: (0, i)),
        out_shape=jax.ShapeDtypeStruct((H, _EP), jnp.float32),
    )(pt, et2d, psum, vg, b8)
    return msg


_SB = 1280
_NSB = _EP // _SB
_CPT = H // _NW


def _scatter_sc(msg, dst, z8, z1):
    mesh = plsc.VectorSubcoreMesh(core_axis_name="c", subcore_axis_name="s")

    @functools.partial(
        pl.kernel, mesh=mesh,
        out_type=jax.ShapeDtypeStruct((_NW * N * _CPT,), jnp.float32),
        scratch_types=[pltpu.VMEM((N,), jnp.float32)] * _CPT + [
            pltpu.VMEM((N,), jnp.float32),
            pltpu.VMEM((_SB,), jnp.int32),
            pltpu.VMEM((_CPT, _SB), jnp.float32),
            pltpu.VMEM((_SB,), jnp.int32),
            pltpu.VMEM((_CPT, _SB), jnp.float32),
            pltpu.SemaphoreType.DMA,
            pltpu.SemaphoreType.DMA,
            pltpu.SemaphoreType.DMA,
            pltpu.SemaphoreType.DMA,
        ],
        compiler_params=pltpu.CompilerParams(needs_layout_passes=False),
    )
    def k(msg_h, dst_h, zA_h, z1_h, agg_h, *bufs):
        wid = lax.axis_index("s") * 2 + lax.axis_index("c")
        aggc = bufs[:_CPT]
        (cntl, dbuf0, mbuf0, dbuf1, mbuf1, sd0, sm0, sd1, sm1) = bufs[_CPT:]
        ones16 = jnp.full((16,), 1.0, jnp.float32)
        for c in range(_CPT):
            pltpu.sync_copy(zA_h.at[pl.ds(0, N)], aggc[c])
        pltpu.sync_copy(z1_h, cntl)

        def refs(b, dbuf, mbuf):
            e0 = b * _SB
            return (dst_h.at[pl.ds(e0, _SB)], dbuf,
                    msg_h.at[pl.ds(wid * _CPT, _CPT), pl.ds(e0, _SB)], mbuf)

        def issue(b, dbuf, mbuf, sd, sm):
            dsrc, ddst, msrc, mdst = refs(b, dbuf, mbuf)
            pltpu.async_copy(dsrc, ddst, sd)
            pltpu.async_copy(msrc, mdst, sm)

        def wait(b, dbuf, mbuf, sd, sm):
            dsrc, ddst, msrc, mdst = refs(b, dbuf, mbuf)
            pltpu.make_async_copy(dsrc, ddst, sd).wait()
            pltpu.make_async_copy(msrc, mdst, sm).wait()

        def process(dbuf, mbuf):
            for g in range(_SB // 16):
                sl = pl.ds(g * 16, 16)
                d16 = dbuf[sl]
                plsc.addupdate_scatter(cntl, [d16], ones16)
                for c in range(_CPT):
                    plsc.addupdate_scatter(aggc[c], [d16], mbuf[c, sl])

        issue(0, dbuf0, mbuf0, sd0, sm0)

        def body(i, _):
            b0 = 2 * i
            issue(b0 + 1, dbuf1, mbuf1, sd1, sm1)
            wait(b0, dbuf0, mbuf0, sd0, sm0)
            process(dbuf0, mbuf0)

            @pl.when(b0 + 2 < _NSB)
            def _issue_next():
                issue(b0 + 2, dbuf0, mbuf0, sd0, sm0)

            wait(b0 + 1, dbuf1, mbuf1, sd1, sm1)
            process(dbuf1, mbuf1)
            return _

        lax.fori_loop(0, _NSB // 2, body, None)

        pad_fix = jnp.where(lax.iota(jnp.int32, 16) == 0,
                            jnp.float32(_EP - E), jnp.float32(0.0))
        cntl[pl.ds(0, 16)] = cntl[pl.ds(0, 16)] - pad_fix

        def fin(j, _):
            sl = pl.ds(j * 16, 16)
            inv = 1.0 / jnp.maximum(cntl[sl], 1.0)
            for c in range(_CPT):
                aggc[c][sl] = aggc[c][sl] * inv
            return _

        lax.fori_loop(0, N // 16, fin, None)
        for c in range(_CPT):
            pltpu.sync_copy(aggc[c], agg_h.at[pl.ds((wid * _CPT + c) * N, N)])

    return k(msg, dst, z8, z1)


_FB = 512


def _final_body(agg_ref, nf_ref, owt_ref, ob_ref, w1t_ref, b1_ref,
                w2t_ref, b2_ref, g1_ref, be1_ref, g2_ref, be2_ref, out_ref):
    acc = jnp.zeros((_FB, H), jnp.float32)
    for t in range(_NW):
        acc = acc + lax.dot_general(agg_ref[t], owt_ref[t],
                                    (((0,), (0,)), ((), ())),
                                    preferred_element_type=jnp.float32)
    x1 = nf_ref[...] + acc + ob_ref[...]
    mu = jnp.mean(x1, axis=1, keepdims=True)
    var = jnp.mean((x1 - mu) ** 2, axis=1, keepdims=True)
    x = (x1 - mu) * lax.rsqrt(var + 1e-5) * g1_ref[...] + be1_ref[...]
    h = jnp.dot(x, w1t_ref[...], preferred_element_type=jnp.float32) + b1_ref[...]
    h = 0.5 * h * (1.0 + lax.erf(h * (2.0 ** -0.5)))
    y = jnp.dot(h, w2t_ref[...], preferred_element_type=jnp.float32) + b2_ref[...]
    x2 = x + y
    mu2 = jnp.mean(x2, axis=1, keepdims=True)
    var2 = jnp.mean((x2 - mu2) ** 2, axis=1, keepdims=True)
    out_ref[...] = (x2 - mu2) * lax.rsqrt(var2 + 1e-5) * g2_ref[...] + be2_ref[...]


def _final(agg, nf, owt, ob, w1t, b1, w2t, b2, g1, be1, g2, be2):
    grid = (pl.cdiv(N, _FB),)
    return pl.pallas_call(
        _final_body,
        grid=grid,
        in_specs=[
            pl.BlockSpec((_NW, _CPT, _FB), lambda i: (0, 0, i)),
            pl.BlockSpec((_FB, H), lambda i: (i, 0)),
            pl.BlockSpec((_NW, _CPT, H), lambda i: (0, 0, 0)),
            pl.BlockSpec((1, H), lambda i: (0, 0)),
            pl.BlockSpec((H, 2 * H), lambda i: (0, 0)),
            pl.BlockSpec((1, 2 * H), lambda i: (0, 0)),
            pl.BlockSpec((2 * H, H), lambda i: (0, 0)),
            pl.BlockSpec((1, H), lambda i: (0, 0)),
            pl.BlockSpec((1, H), lambda i: (0, 0)),
            pl.BlockSpec((1, H), lambda i: (0, 0)),
            pl.BlockSpec((1, H), lambda i: (0, 0)),
            pl.BlockSpec((1, H), lambda i: (0, 0)),
        ],
        out_specs=pl.BlockSpec((_FB, H), lambda i: (i, 0)),
        out_shape=jax.ShapeDtypeStruct((N, H), jnp.float32),
    )(agg, nf, owt, ob, w1t, b1, w2t, b2, g1, be1, g2, be2)


def kernel(node_feats, edge_index, edge_type, qW, qb, kW, kb, vW, vb,
           oW, ob, w1, b1, w2, b2, g1, be1, g2, be2):
    scale = HD ** (-0.5)
    src = edge_index[0]
    dst = edge_index[1]
    et = edge_type
    zpad = jnp.zeros((_EP - E,), jnp.int32)
    srcp = jnp.concatenate([src, zpad])
    dstp = jnp.concatenate([dst, zpad])
    etp0 = jnp.concatenate([et, zpad])
    et2d = jnp.concatenate([et, zpad + R]).reshape(1, _EP)

    wq = qW.T * scale
    bq = (qb * scale).reshape(1, H)
    wk = jnp.transpose(kW, (2, 0, 1)).reshape(H, R * H)
    wv = jnp.transpose(vW, (2, 0, 1)).reshape(H, R * H)
    bk = kb.reshape(1, R * H)
    bv = vb.reshape(1, R * H)
    b8 = (jnp.arange(H, dtype=jnp.int32)[None, :] // HD ==
          jnp.arange(NH, dtype=jnp.int32)[:, None]).astype(jnp.float32)
    owt = oW.T.reshape(_NW, _CPT, H)
    w1t = w1.T
    w2t = w2.T
    z8 = jnp.zeros((N,), jnp.float32)
    z1 = jnp.zeros((N,), jnp.float32)

    qtab, ktab2, vtab2 = _projections(node_feats, wq, wk, wv, bq, bk, bv)
    ktab = ktab2.reshape(N * R, H)
    vtab = vtab2.reshape(N * R, H)

    qg, kg, vg = _gather_sc(qtab, ktab, vtab, srcp, dstp, etp0)
    msg = _scores_and_messages(qg, kg, vg, et2d, b8)
    agg = _scatter_sc(msg, dstp, z8, z1).reshape(_NW, _CPT, N)

    return _final(agg, node_feats, owt, ob.reshape(1, H), w1t,
                  b1.reshape(1, 2 * H), w2t, b2.reshape(1, H),
                  g1.reshape(1, H), be1.reshape(1, H),
                  g2.reshape(1, H), be2.reshape(1, H))


# speedup vs baseline: 1.1985x; 1.0396x over previous
import functools

import jax
import jax.numpy as jnp
from jax import lax
from jax.experimental import pallas as pl
from jax.experimental.pallas import tpu as pltpu
from jax.experimental.pallas import tpu_sc as plsc

N = 10000
E = 160000
H = 256
R = 8
NH = 8
HD = H // NH


_RB = 1024


def _proj_body(x_ref, wq_ref, wk_ref, wv_ref, bq_ref, bk_ref, bv_ref,
               q_ref, k_ref, v_ref):
    x = x_ref[...]
    q_ref[...] = jnp.dot(x, wq_ref[...], preferred_element_type=jnp.float32) + bq_ref[...]
    k_ref[...] = jnp.dot(x, wk_ref[...], preferred_element_type=jnp.float32) + bk_ref[...]
    v_ref[...] = jnp.dot(x, wv_ref[...], preferred_element_type=jnp.float32) + bv_ref[...]


def _projections(x, wq, wk, wv, bq, bk, bv):
    grid = (pl.cdiv(N, _RB),)
    return pl.pallas_call(
        _proj_body,
        grid=grid,
        in_specs=[
            pl.BlockSpec((_RB, H), lambda i: (i, 0)),
            pl.BlockSpec((H, H), lambda i: (0, 0)),
            pl.BlockSpec((H, R * H), lambda i: (0, 0)),
            pl.BlockSpec((H, R * H), lambda i: (0, 0)),
            pl.BlockSpec((1, H), lambda i: (0, 0)),
            pl.BlockSpec((1, R * H), lambda i: (0, 0)),
            pl.BlockSpec((1, R * H), lambda i: (0, 0)),
        ],
        out_specs=[
            pl.BlockSpec((_RB, H), lambda i: (i, 0)),
            pl.BlockSpec((_RB, R * H), lambda i: (i, 0)),
            pl.BlockSpec((_RB, R * H), lambda i: (i, 0)),
        ],
        out_shape=[
            jax.ShapeDtypeStruct((N, H), jnp.float32),
            jax.ShapeDtypeStruct((N, R * H), jnp.float32),
            jax.ShapeDtypeStruct((N, R * H), jnp.float32),
        ],
    )(x, wq, wk, wv, bq, bk, bv)


_GB = 128
_NW = 32
_EP = 163840
_NGB = _EP // _GB
_GIT = _NGB // _NW


def _gather_sc(qtab, ktab, vtab, src, dst, et):
    mesh = plsc.VectorSubcoreMesh(core_axis_name="c", subcore_axis_name="s")

    @functools.partial(
        pl.kernel, mesh=mesh,
        out_type=[
            jax.ShapeDtypeStruct((_EP, H), jnp.float32),
            jax.ShapeDtypeStruct((_EP, H), jnp.float32),
            jax.ShapeDtypeStruct((_EP, H), jnp.float32),
        ],
        scratch_types=[
            pltpu.VMEM((_GB,), jnp.int32),
            pltpu.VMEM((_GB,), jnp.int32),
            pltpu.VMEM((_GB,), jnp.int32),
            pltpu.VMEM((_GB,), jnp.int32),
            pltpu.VMEM((_GB, H), jnp.float32),
            pltpu.VMEM((_GB, H), jnp.float32),
            pltpu.VMEM((_GB, H), jnp.float32),
            pltpu.SemaphoreType.DMA,
            pltpu.SemaphoreType.DMA,
            pltpu.SemaphoreType.DMA,
            pltpu.SemaphoreType.DMA,
        ],
        compiler_params=pltpu.CompilerParams(needs_layout_passes=False),
    )
    def k(qtab_h, ktab_h, vtab_h, src_h, dst_h, et_h, qg_h, kg_h, vg_h,
          sbuf, dbuf, ebuf, kibuf, qstage, kstage, vstage,
          sem0, sem1, sem2, semw):
        wid = lax.axis_index("s") * 2 + lax.axis_index("c")

        def wb_refs(e0):
            return ((qstage, qg_h.at[pl.ds(e0, _GB)]),
                    (kstage, kg_h.at[pl.ds(e0, _GB)]),
                    (vstage, vg_h.at[pl.ds(e0, _GB)]))

        def body(i, _):
            b = wid + i * _NW
            e0 = b * _GB
            pltpu.sync_copy(src_h.at[pl.ds(e0, _GB)], sbuf)
            pltpu.sync_copy(dst_h.at[pl.ds(e0, _GB)], dbuf)
            pltpu.sync_copy(et_h.at[pl.ds(e0, _GB)], ebuf)
            for g in range(_GB // 16):
                sl = pl.ds(g * 16, 16)
                kibuf[sl] = sbuf[sl] * R + ebuf[sl]

            @pl.when(i > 0)
            def _():
                for s, d in wb_refs(e0):
                    pltpu.make_async_copy(s, d, semw).wait()

            cq = pltpu.async_copy(qtab_h.at[dbuf], qstage, sem0)
            ck = pltpu.async_copy(ktab_h.at[kibuf], kstage, sem1)
            cv = pltpu.async_copy(vtab_h.at[kibuf], vstage, sem2)
            cq.wait()
            ck.wait()
            cv.wait()
            for s, d in wb_refs(e0):
                pltpu.async_copy(s, d, semw)
            return _

        lax.fori_loop(0, _GIT, body, None)
        for s, d in wb_refs((wid + (_GIT - 1) * _NW) * _GB):
            pltpu.make_async_copy(s, d, semw).wait()

    return k(qtab, ktab, vtab, src, dst, et)


_EB = 5120
_NEB = _EP // _EB
_NEG = -1e30


def _s1_body(qg_ref, kg_ref, et_ref, b8_ref, st_ref, pmax_ref):
    qk = qg_ref[...] * kg_ref[...]
    st = lax.dot_general(b8_ref[...], qk, (((1,), (1,)), ((), ())),
                         preferred_element_type=jnp.float32)
    st_ref[...] = st
    et = et_ref[...]
    for r in range(R):
        m = jnp.max(jnp.where(et == r, st, _NEG), axis=1)
        pmax_ref[0, r, :] = m


def _s2_body(st_ref, et_ref, pmax_ref, pt_ref, psum_ref):
    mx = jnp.max(pmax_ref[...], axis=0)
    et = et_ref[...]
    st = st_ref[...]
    msel = jnp.zeros_like(st)
    for r in range(R):
        msel = jnp.where(et == r, mx[r].reshape(NH, 1), msel)
    p = jnp.exp(st - msel)
    pt_ref[...] = p
    for r in range(R):
        s = jnp.sum(jnp.where(et == r, p, 0.0), axis=1)
        psum_ref[0, r, :] = s


def _s3_body(pt_ref, et_ref, psum_ref, vg_ref, b8_ref, msg_ref):
    denom = jnp.sum(psum_ref[...], axis=0)
    inv = 1.0 / denom
    et = et_ref[...]
    p = pt_ref[...]
    isel = jnp.zeros_like(p)
    for r in range(R):
        isel = jnp.where(et == r, inv[r].reshape(NH, 1), isel)
    wt = p * isel
    wide = lax.dot_general(wt, b8_ref[...], (((0,), (0,)), ((), ())),
                           preferred_element_type=jnp.float32)
    msg_ref[...] = jnp.transpose(wide * vg_ref[...])


def _scores_and_messages(qg, kg, vg, et2d, b8):
    st, pmax = pl.pallas_call(
        _s1_body,
        grid=(_NEB,),
        in_specs=[
            pl.BlockSpec((_EB, H), lambda i: (i, 0)),
            pl.BlockSpec((_EB, H), lambda i: (i, 0)),
            pl.BlockSpec((1, _EB), lambda i: (0, i)),
            pl.BlockSpec((NH, H), lambda i: (0, 0)),
        ],
        out_specs=[
            pl.BlockSpec((NH, _EB), lambda i: (0, i)),
            pl.BlockSpec((1, R, NH), lambda i: (i, 0, 0)),
        ],
        out_shape=[
            jax.ShapeDtypeStruct((NH, _EP), jnp.float32),
            jax.ShapeDtypeStruct((_NEB, R, NH), jnp.float32),
        ],
    )(qg, kg, et2d, b8)

    pt, psum = pl.pallas_call(
        _s2_body,
        grid=(_NEB,),
        in_specs=[
            pl.BlockSpec((NH, _EB), lambda i: (0, i)),
            pl.BlockSpec((1, _EB), lambda i: (0, i)),
            pl.BlockSpec((_NEB, R, NH), lambda i: (0, 0, 0)),
        ],
        out_specs=[
            pl.BlockSpec((NH, _EB), lambda i: (0, i)),
            pl.BlockSpec((1, R, NH), lambda i: (i, 0, 0)),
        ],
        out_shape=[
            jax.ShapeDtypeStruct((NH, _EP), jnp.float32),
            jax.ShapeDtypeStruct((_NEB, R, NH), jnp.float32),
        ],
    )(st, et2d, pmax)

    msg = pl.pallas_call(
        _s3_body,
        grid=(_NEB,),
        in_specs=[
            pl.BlockSpec((NH, _EB), lambda i: (0, i)),
            pl.BlockSpec((1, _EB), lambda i: (0, i)),
            pl.BlockSpec((_NEB, R, NH), lambda i: (0, 0, 0)),
            pl.BlockSpec((_EB, H), lambda i: (i, 0)),
            pl.BlockSpec((NH, H), lambda i: (0, 0)),
        ],
        out_specs=pl.BlockSpec((H, _EB), lambda i: (0, i)),
        out_shape=jax.ShapeDtypeStruct((H, _EP), jnp.float32),
    )(pt, et2d, psum, vg, b8)
    return msg


_SB = 1280
_NSB = _EP // _SB
_CPT = H // _NW


def _scatter_sc(msg, dst, z8, z1):
    mesh = plsc.VectorSubcoreMesh(core_axis_name="c", subcore_axis_name="s")

    @functools.partial(
        pl.kernel, mesh=mesh,
        out_type=jax.ShapeDtypeStruct((_NW * N * _CPT,), jnp.float32),
        scratch_types=[pltpu.VMEM((N,), jnp.float32)] * _CPT + [
            pltpu.VMEM((N,), jnp.float32),
            pltpu.VMEM((_SB,), jnp.int32),
            pltpu.VMEM((_CPT, _SB), jnp.float32),
            pltpu.VMEM((_SB,), jnp.int32),
            pltpu.VMEM((_CPT, _SB), jnp.float32),
            pltpu.SemaphoreType.DMA,
            pltpu.SemaphoreType.DMA,
            pltpu.SemaphoreType.DMA,
            pltpu.SemaphoreType.DMA,
        ],
        compiler_params=pltpu.CompilerParams(needs_layout_passes=False),
    )
    def k(msg_h, dst_h, zA_h, z1_h, agg_h, *bufs):
        wid = lax.axis_index("s") * 2 + lax.axis_index("c")
        aggc = bufs[:_CPT]
        (cntl, dbuf0, mbuf0, dbuf1, mbuf1, sd0, sm0, sd1, sm1) = bufs[_CPT:]
        ones16 = jnp.full((16,), 1.0, jnp.float32)
        for c in range(_CPT):
            pltpu.sync_copy(zA_h.at[pl.ds(0, N)], aggc[c])
        pltpu.sync_copy(z1_h, cntl)

        def refs(b, dbuf, mbuf):
            e0 = b * _SB
            return (dst_h.at[pl.ds(e0, _SB)], dbuf,
                    msg_h.at[pl.ds(wid * _CPT, _CPT), pl.ds(e0, _SB)], mbuf)

        def issue(b, dbuf, mbuf, sd, sm):
            dsrc, ddst, msrc, mdst = refs(b, dbuf, mbuf)
            pltpu.async_copy(dsrc, ddst, sd)
            pltpu.async_copy(msrc, mdst, sm)

        def wait(b, dbuf, mbuf, sd, sm):
            dsrc, ddst, msrc, mdst = refs(b, dbuf, mbuf)
            pltpu.make_async_copy(dsrc, ddst, sd).wait()
            pltpu.make_async_copy(msrc, mdst, sm).wait()

        def process(dbuf, mbuf):
            for g in range(_SB // 16):
                sl = pl.ds(g * 16, 16)
                d16 = dbuf[sl]
                plsc.addupdate_scatter(cntl, [d16], ones16)
                for c in range(_CPT):
                    plsc.addupdate_scatter(aggc[c], [d16], mbuf[c, sl])

        issue(0, dbuf0, mbuf0, sd0, sm0)

        def body(i, _):
            b0 = 2 * i
            issue(b0 + 1, dbuf1, mbuf1, sd1, sm1)
            wait(b0, dbuf0, mbuf0, sd0, sm0)
            process(dbuf0, mbuf0)

            @pl.when(b0 + 2 < _NSB)
            def _issue_next():
                issue(b0 + 2, dbuf0, mbuf0, sd0, sm0)

            wait(b0 + 1, dbuf1, mbuf1, sd1, sm1)
            process(dbuf1, mbuf1)
            return _

        lax.fori_loop(0, _NSB // 2, body, None)

        pad_fix = jnp.where(lax.iota(jnp.int32, 16) == 0,
                            jnp.float32(_EP - E), jnp.float32(0.0))
        cntl[pl.ds(0, 16)] = cntl[pl.ds(0, 16)] - pad_fix

        def fin(j, _):
            sl = pl.ds(j * 16, 16)
            inv = 1.0 / jnp.maximum(cntl[sl], 1.0)
            for c in range(_CPT):
                aggc[c][sl] = aggc[c][sl] * inv
            return _

        lax.fori_loop(0, N // 16, fin, None)
        for c in range(_CPT):
            pltpu.sync_copy(aggc[c], agg_h.at[pl.ds((wid * _CPT + c) * N, N)])

    return k(msg, dst, z8, z1)


_FB = 512


def _final_body(agg_ref, nf_ref, owt_ref, ob_ref, w1t_ref, b1_ref,
                w2t_ref, b2_ref, g1_ref, be1_ref, g2_ref, be2_ref, out_ref):
    acc = jnp.zeros((_FB, H), jnp.float32)
    for t in range(_NW):
        acc = acc + lax.dot_general(agg_ref[t], owt_ref[t],
                                    (((0,), (0,)), ((), ())),
                                    preferred_element_type=jnp.float32)
    x1 = nf_ref[...] + acc + ob_ref[...]
    mu = jnp.mean(x1, axis=1, keepdims=True)
    var = jnp.mean((x1 - mu) ** 2, axis=1, keepdims=True)
    x = (x1 - mu) * lax.rsqrt(var + 1e-5) * g1_ref[...] + be1_ref[...]
    h = jnp.dot(x, w1t_ref[...], preferred_element_type=jnp.float32) + b1_ref[...]
    h = 0.5 * h * (1.0 + lax.erf(h * (2.0 ** -0.5)))
    y = jnp.dot(h, w2t_ref[...], preferred_element_type=jnp.float32) + b2_ref[...]
    x2 = x + y
    mu2 = jnp.mean(x2, axis=1, keepdims=True)
    var2 = jnp.mean((x2 - mu2) ** 2, axis=1, keepdims=True)
    out_ref[...] = (x2 - mu2) * lax.rsqrt(var2 + 1e-5) * g2_ref[...] + be2_ref[...]


def _final(agg, nf, owt, ob, w1t, b1, w2t, b2, g1, be1, g2, be2):
    grid = (pl.cdiv(N, _FB),)
    return pl.pallas_call(
        _final_body,
        grid=grid,
        in_specs=[
            pl.BlockSpec((_NW, _CPT, _FB), lambda i: (0, 0, i)),
            pl.BlockSpec((_FB, H), lambda i: (i, 0)),
            pl.BlockSpec((_NW, _CPT, H), lambda i: (0, 0, 0)),
            pl.BlockSpec((1, H), lambda i: (0, 0)),
            pl.BlockSpec((H, 2 * H), lambda i: (0, 0)),
            pl.BlockSpec((1, 2 * H), lambda i: (0, 0)),
            pl.BlockSpec((2 * H, H), lambda i: (0, 0)),
            pl.BlockSpec((1, H), lambda i: (0, 0)),
            pl.BlockSpec((1, H), lambda i: (0, 0)),
            pl.BlockSpec((1, H), lambda i: (0, 0)),
            pl.BlockSpec((1, H), lambda i: (0, 0)),
            pl.BlockSpec((1, H), lambda i: (0, 0)),
        ],
        out_specs=pl.BlockSpec((_FB, H), lambda i: (i, 0)),
        out_shape=jax.ShapeDtypeStruct((N, H), jnp.float32),
    )(agg, nf, owt, ob, w1t, b1, w2t, b2, g1, be1, g2, be2)


def kernel(node_feats, edge_index, edge_type, qW, qb, kW, kb, vW, vb,
           oW, ob, w1, b1, w2, b2, g1, be1, g2, be2):
    scale = HD ** (-0.5)
    src = edge_index[0]
    dst = edge_index[1]
    et = edge_type
    zpad = jnp.zeros((_EP - E,), jnp.int32)
    srcp = jnp.concatenate([src, zpad])
    dstp = jnp.concatenate([dst, zpad])
    etp0 = jnp.concatenate([et, zpad])
    et2d = jnp.concatenate([et, zpad + R]).reshape(1, _EP)

    wq = qW.T * scale
    bq = (qb * scale).reshape(1, H)
    wk = jnp.transpose(kW, (2, 0, 1)).reshape(H, R * H)
    wv = jnp.transpose(vW, (2, 0, 1)).reshape(H, R * H)
    bk = kb.reshape(1, R * H)
    bv = vb.reshape(1, R * H)
    b8 = (jnp.arange(H, dtype=jnp.int32)[None, :] // HD ==
          jnp.arange(NH, dtype=jnp.int32)[:, None]).astype(jnp.float32)
    owt = oW.T.reshape(_NW, _CPT, H)
    w1t = w1.T
    w2t = w2.T
    z8 = jnp.zeros((N,), jnp.float32)
    z1 = jnp.zeros((N,), jnp.float32)

    qtab, ktab2, vtab2 = _projections(node_feats, wq, wk, wv, bq, bk, bv)
    ktab = ktab2.reshape(N * R, H)
    vtab = vtab2.reshape(N * R, H)

    qg, kg, vg = _gather_sc(qtab, ktab, vtab, srcp, dstp, etp0)
    msg = _scores_and_messages(qg, kg, vg, et2d, b8)
    agg = _scatter_sc(msg, dstp, z8, z1).reshape(_NW, _CPT, N)

    return _final(agg, node_feats, owt, ob.reshape(1, H), w1t,
                  b1.reshape(1, 2 * H), w2t, b2.reshape(1, H),
                  g1.reshape(1, H), be1.reshape(1, H),
                  g2.reshape(1, H), be2.reshape(1, H))
